# baseline (speedup 1.0000x reference)
import functools

import numpy as np
import jax
import jax.numpy as jnp
from jax import lax
from jax.experimental import pallas as pl
from jax.experimental.pallas import tpu as pltpu
from jax.experimental.pallas import tpu_sc as plsc

SEQ_LEN = 8192
BATCH = 128
N_ODD = BATCH // 2
CHUNKS = SEQ_LEN // 16

_U32MAX = np.iinfo(np.uint32).max


def _one_round_max() -> int:
    def rounds(size):
        return int(np.ceil(3 * np.log(max(1, size)) / np.log(_U32MAX)))

    lo, hi = 1, SEQ_LEN
    while lo < hi:
        mid = (lo + hi + 1) // 2
        if rounds(mid) <= 1:
            lo = mid
        else:
            hi = mid - 1
    return lo


ORM = _one_round_max()


def _tf_block(k0, k1, c0, c1):
    k0 = np.uint32(k0)
    k1 = np.uint32(k1)
    ks0, ks1 = k0, k1
    ks2 = k0 ^ k1 ^ np.uint32(0x1BD11BDA)
    c0 = np.asarray(c0, np.uint32)
    c1 = np.asarray(c1, np.uint32)

    def rotl(x, d):
        d = np.uint32(d)
        return (x << d) | (x >> np.uint32(32 - d))

    def four(x0, x1, rots):
        for d in rots:
            x0 = x0 + x1
            x1 = rotl(x1, d)
            x1 = x1 ^ x0
        return x0, x1

    rot_a = (13, 15, 26, 6)
    rot_b = (17, 29, 16, 24)
    with np.errstate(over="ignore"):
        x0 = c0 + ks0
        x1 = c1 + ks1
        x0, x1 = four(x0, x1, rot_a)
        x0 = x0 + ks1
        x1 = x1 + ks2 + np.uint32(1)
        x0, x1 = four(x0, x1, rot_b)
        x0 = x0 + ks2
        x1 = x1 + ks0 + np.uint32(2)
        x0, x1 = four(x0, x1, rot_a)
        x0 = x0 + ks0
        x1 = x1 + ks1 + np.uint32(3)
        x0, x1 = four(x0, x1, rot_b)
        x0 = x0 + ks1
        x1 = x1 + ks2 + np.uint32(4)
        x0, x1 = four(x0, x1, rot_a)
        x0 = x0 + ks2
        x1 = x1 + ks0 + np.uint32(5)
    return x0, x1


def _fold_in(k, i):
    o0, o1 = _tf_block(k[0], k[1], np.uint32(0), np.uint32(i))
    return o0, o1


def _split2(k):
    o0, o1 = _tf_block(k[0], k[1], np.zeros(2, np.uint32), np.arange(2, dtype=np.uint32))
    return (o0[0], o1[0]), (o0[1], o1[1])


def _bits_row(k):
    c1 = np.arange(SEQ_LEN, dtype=np.uint32)
    o0, o1 = _tf_block(k[0], k[1], np.zeros(SEQ_LEN, np.uint32), c1)
    return o0 ^ o1


def _build_tables():
    sig1 = np.empty((N_ODD, SEQ_LEN), np.int32)
    sig2 = np.empty((N_ODD, SEQ_LEN), np.int32)
    for j in range(N_ODD):
        i = 2 * j + 1
        k = _fold_in((np.uint32(0), np.uint32(42)), i)
        k, sub1 = _split2(k)
        k, sub2 = _split2(k)
        sig1[j] = np.argsort(_bits_row(sub1), kind="stable")
        sig2[j] = np.argsort(_bits_row(sub2), kind="stable")
    return sig1, sig2


_SIG1_NP, _SIG2_NP = _build_tables()


_MESH = plsc.VectorSubcoreMesh(core_axis_name="c", subcore_axis_name="s")

W2 = 2 * SEQ_LEN
WCHUNKS = (W2 + 32) // 16


@functools.partial(
    pl.kernel,
    mesh=_MESH,
    compiler_params=pltpu.CompilerParams(needs_layout_passes=False),
    out_type=jax.ShapeDtypeStruct((BATCH, W2), jnp.int32),
    scratch_types=[
        pltpu.VMEM((W2,), jnp.int32),
        pltpu.VMEM((SEQ_LEN,), jnp.int32),
        pltpu.VMEM((SEQ_LEN,), jnp.int32),
        pltpu.VMEM((SEQ_LEN + 16,), jnp.int32),
        pltpu.VMEM((SEQ_LEN + 16,), jnp.int32),
        pltpu.VMEM((W2 + 32,), jnp.int32),
        pltpu.SemaphoreType.DMA,
        pltpu.SemaphoreType.DMA,
    ],
)
def _dropout_sc(
    x_hbm, sig1_hbm, sig2_hbm, out_hbm, seq_v, s1_v, s2_v, a_v, b_v, o_v, sem_e1, sem_e2
):
    wid = lax.axis_index("s") * jnp.int32(2) + lax.axis_index("c")

    zeros16 = jnp.zeros((16,), jnp.int32)
    ones16 = jnp.full((16,), 1, jnp.int32)
    lanes16 = lax.iota(jnp.int32, 16)

    e1 = wid * jnp.int32(2)
    e2 = e1 + jnp.int32(64)
    pltpu.sync_copy(x_hbm.at[e1], seq_v)
    pltpu.sync_copy(seq_v, out_hbm.at[e1])
    pltpu.sync_copy(x_hbm.at[e2], seq_v)
    pltpu.sync_copy(seq_v, out_hbm.at[e2])
    del sem_e1, sem_e2

    @plsc.parallel_loop(0, WCHUNKS, unroll=4)
    def _z0(t):
        o_v[pl.ds(t * jnp.int32(16), 16)] = zeros16

    def do_row(j):
        gr = jnp.int32(2) * j + jnp.int32(1)
        pltpu.sync_copy(x_hbm.at[gr], seq_v)
        pltpu.sync_copy(sig1_hbm.at[j], s1_v)
        pltpu.sync_copy(sig2_hbm.at[j], s2_v)

        @plsc.parallel_loop(0, W2 // 16, carry=zeros16, unroll=4)
        def n_splat(t, acc):
            v = seq_v[pl.ds(t * jnp.int32(16), 16)]
            return acc + plsc.all_reduce_population_count(v > 0)

        n_scal = jnp.max(n_splat)
        d_splat = n_splat // jnp.int32(10)
        d_scal = n_scal // jnp.int32(10)

        @plsc.parallel_loop(0, CHUNKS, carry=(zeros16, zeros16), unroll=4)
        def _ab(t, cs):
            ca, cb = cs
            off = t * jnp.int32(16)
            v1 = s1_v[pl.ds(off, 16)]
            v2 = s2_v[pl.ds(off, 16)]
            m1 = v1 < n_splat
            m2 = v2 < n_splat
            o1 = jnp.where(m1, ones16, zeros16)
            o2 = jnp.where(m2, ones16, zeros16)
            i1 = plsc.cumsum(o1)
            i2 = plsc.cumsum(o2)
            plsc.store_scatter(a_v, [ca + i1 - o1], v1, mask=m1)
            plsc.store_scatter(b_v, [cb + i2 - o2], v2, mask=m2)
            return (
                ca + plsc.all_reduce_population_count(m1),
                cb + plsc.all_reduce_population_count(m2),
            )

        @pl.when(d_scal > 0)
        def _():
            use2 = n_splat > jnp.int32(ORM)

            @plsc.parallel_loop(0, (d_scal + jnp.int32(15)) // jnp.int32(16), unroll=4)
            def _drop(mi):
                lanes = lanes16 + mi * jnp.int32(16)
                msk = lanes < d_splat
                bidx = b_v[pl.ds(mi * jnp.int32(16), 16)]
                sel = jnp.where(use2, bidx, lanes)
                sel = jnp.where(msk, sel, zeros16)
                pidx = plsc.load_gather(a_v, [sel])
                pidx = jnp.where(msk, pidx, zeros16)
                plsc.store_scatter(seq_v, [pidx * jnp.int32(2)], zeros16, mask=msk)

            @plsc.parallel_loop(0, W2 // 16, carry=zeros16, unroll=4)
            def c_splat(t, cs):
                v = seq_v[pl.ds(t * jnp.int32(16), 16)]
                m = v > 0
                o = jnp.where(m, ones16, zeros16)
                i = plsc.cumsum(o)
                pos = (cs + i - o) * jnp.int32(2)
                plsc.store_scatter(o_v, [pos], v, mask=m)
                return cs + plsc.all_reduce_population_count(m)

            w_scal = jnp.int32(2) * jnp.max(c_splat)
            o_v[pl.ds(w_scal, 16)] = zeros16
            o_v[pl.ds(w_scal + jnp.int32(16), 16)] = zeros16

            @plsc.parallel_loop(
                (w_scal + jnp.int32(31)) // jnp.int32(16), W2 // 16, unroll=4
            )
            def _ztail(t):
                o_v[pl.ds(t * jnp.int32(16), 16)] = zeros16

            pltpu.sync_copy(o_v.at[pl.ds(0, W2)], out_hbm.at[gr])

        @pl.when(d_scal == 0)
        def _():
            pltpu.sync_copy(seq_v, out_hbm.at[gr])

    do_row(wid)
    do_row(wid + jnp.int32(32))


def kernel(input_ids):
    x32 = lax.bitcast_convert_type(input_ids, jnp.int32).reshape(BATCH, W2)
    with jax.enable_x64(False):
        out32 = _dropout_sc(x32, jnp.asarray(_SIG1_NP), jnp.asarray(_SIG2_NP))
    return lax.bitcast_convert_type(
        out32.reshape(BATCH, SEQ_LEN, 2), jnp.int64
    )


# speedup vs baseline: 1.1706x; 1.1706x over previous
import functools

import numpy as np
import jax
import jax.numpy as jnp
from jax import lax
from jax.experimental import pallas as pl
from jax.experimental.pallas import tpu as pltpu
from jax.experimental.pallas import tpu_sc as plsc

SEQ_LEN = 8192
BATCH = 128
N_ODD = BATCH // 2
CHUNKS = SEQ_LEN // 16

_U32MAX = np.iinfo(np.uint32).max


def _one_round_max() -> int:
    def rounds(size):
        return int(np.ceil(3 * np.log(max(1, size)) / np.log(_U32MAX)))

    lo, hi = 1, SEQ_LEN
    while lo < hi:
        mid = (lo + hi + 1) // 2
        if rounds(mid) <= 1:
            lo = mid
        else:
            hi = mid - 1
    return lo


ORM = _one_round_max()


def _tf_block(k0, k1, c0, c1):
    k0 = np.uint32(k0)
    k1 = np.uint32(k1)
    ks0, ks1 = k0, k1
    ks2 = k0 ^ k1 ^ np.uint32(0x1BD11BDA)
    c0 = np.asarray(c0, np.uint32)
    c1 = np.asarray(c1, np.uint32)

    def rotl(x, d):
        d = np.uint32(d)
        return (x << d) | (x >> np.uint32(32 - d))

    def four(x0, x1, rots):
        for d in rots:
            x0 = x0 + x1
            x1 = rotl(x1, d)
            x1 = x1 ^ x0
        return x0, x1

    rot_a = (13, 15, 26, 6)
    rot_b = (17, 29, 16, 24)
    with np.errstate(over="ignore"):
        x0 = c0 + ks0
        x1 = c1 + ks1
        x0, x1 = four(x0, x1, rot_a)
        x0 = x0 + ks1
        x1 = x1 + ks2 + np.uint32(1)
        x0, x1 = four(x0, x1, rot_b)
        x0 = x0 + ks2
        x1 = x1 + ks0 + np.uint32(2)
        x0, x1 = four(x0, x1, rot_a)
        x0 = x0 + ks0
        x1 = x1 + ks1 + np.uint32(3)
        x0, x1 = four(x0, x1, rot_b)
        x0 = x0 + ks1
        x1 = x1 + ks2 + np.uint32(4)
        x0, x1 = four(x0, x1, rot_a)
        x0 = x0 + ks2
        x1 = x1 + ks0 + np.uint32(5)
    return x0, x1


def _fold_in(k, i):
    o0, o1 = _tf_block(k[0], k[1], np.uint32(0), np.uint32(i))
    return o0, o1


def _split2(k):
    o0, o1 = _tf_block(k[0], k[1], np.zeros(2, np.uint32), np.arange(2, dtype=np.uint32))
    return (o0[0], o1[0]), (o0[1], o1[1])


def _bits_row(k):
    c1 = np.arange(SEQ_LEN, dtype=np.uint32)
    o0, o1 = _tf_block(k[0], k[1], np.zeros(SEQ_LEN, np.uint32), c1)
    return o0 ^ o1


def _build_tables():
    sig1 = np.empty((N_ODD, SEQ_LEN), np.int32)
    sig2 = np.empty((N_ODD, SEQ_LEN), np.int32)
    for j in range(N_ODD):
        i = 2 * j + 1
        k = _fold_in((np.uint32(0), np.uint32(42)), i)
        k, sub1 = _split2(k)
        k, sub2 = _split2(k)
        sig1[j] = np.argsort(_bits_row(sub1), kind="stable")
        sig2[j] = np.argsort(_bits_row(sub2), kind="stable")
    return sig1, sig2


_SIG1_NP, _SIG2_NP = _build_tables()


_MESH = plsc.VectorSubcoreMesh(core_axis_name="c", subcore_axis_name="s")


@functools.partial(
    pl.kernel,
    mesh=_MESH,
    compiler_params=pltpu.CompilerParams(needs_layout_passes=False),
    out_type=jax.ShapeDtypeStruct((N_ODD, SEQ_LEN), jnp.int32),
    scratch_types=[
        pltpu.VMEM((SEQ_LEN,), jnp.int32),
        pltpu.VMEM((SEQ_LEN,), jnp.int32),
        pltpu.VMEM((SEQ_LEN,), jnp.int32),
        pltpu.VMEM((SEQ_LEN + 16,), jnp.int32),
        pltpu.VMEM((SEQ_LEN + 16,), jnp.int32),
        pltpu.VMEM((SEQ_LEN + 16,), jnp.int32),
    ],
)
def _dropout_sc(x_hbm, sig1_hbm, sig2_hbm, out_hbm, seq_v, s1_v, s2_v, a_v, b_v, o_v):
    wid = lax.axis_index("s") * jnp.int32(2) + lax.axis_index("c")

    zeros16 = jnp.zeros((16,), jnp.int32)
    lanes16 = lax.iota(jnp.int32, 16)

    ones16 = jnp.full((16,), 1, jnp.int32)

    def do_row(r):
        pltpu.sync_copy(x_hbm.at[r], seq_v)
        pltpu.sync_copy(sig1_hbm.at[r], s1_v)
        pltpu.sync_copy(sig2_hbm.at[r], s2_v)

        @plsc.parallel_loop(0, CHUNKS, carry=zeros16, unroll=4)
        def n_splat(t, acc):
            v = seq_v[pl.ds(t * jnp.int32(16), 16)]
            return acc + plsc.all_reduce_population_count(v > 0)

        n_scal = jnp.max(n_splat)
        d_splat = n_splat // jnp.int32(10)
        d_scal = n_scal // jnp.int32(10)

        @plsc.parallel_loop(0, CHUNKS, carry=(zeros16, zeros16), unroll=4)
        def _ab(t, cs):
            ca, cb = cs
            off = t * jnp.int32(16)
            v1 = s1_v[pl.ds(off, 16)]
            v2 = s2_v[pl.ds(off, 16)]
            m1 = v1 < n_splat
            m2 = v2 < n_splat
            o1 = jnp.where(m1, ones16, zeros16)
            o2 = jnp.where(m2, ones16, zeros16)
            i1 = plsc.cumsum(o1)
            i2 = plsc.cumsum(o2)
            plsc.store_scatter(a_v, [ca + i1 - o1], v1, mask=m1)
            plsc.store_scatter(b_v, [cb + i2 - o2], v2, mask=m2)
            return (
                ca + plsc.all_reduce_population_count(m1),
                cb + plsc.all_reduce_population_count(m2),
            )

        @pl.when(d_scal > 0)
        def _():
            use2 = n_splat > jnp.int32(ORM)

            @plsc.parallel_loop(0, (d_scal + jnp.int32(15)) // jnp.int32(16), unroll=4)
            def _drop(mi):
                lanes = lanes16 + mi * jnp.int32(16)
                msk = lanes < d_splat
                bidx = b_v[pl.ds(mi * jnp.int32(16), 16)]
                sel = jnp.where(use2, bidx, lanes)
                sel = jnp.where(msk, sel, zeros16)
                pidx = plsc.load_gather(a_v, [sel])
                pidx = jnp.where(msk, pidx, zeros16)
                plsc.store_scatter(seq_v, [pidx], zeros16, mask=msk)

            @plsc.parallel_loop(0, CHUNKS, carry=zeros16, unroll=4)
            def c_splat(t, cs):
                v = seq_v[pl.ds(t * jnp.int32(16), 16)]
                m = v > 0
                o = jnp.where(m, ones16, zeros16)
                i = plsc.cumsum(o)
                plsc.store_scatter(o_v, [cs + i - o], v, mask=m)
                return cs + plsc.all_reduce_population_count(m)

            c_scal = jnp.max(c_splat)
            plsc.store_scatter(o_v, [c_splat + lanes16], zeros16)

            @plsc.parallel_loop((c_scal + jnp.int32(15)) // jnp.int32(16), CHUNKS, unroll=4)
            def _ztail(t):
                o_v[pl.ds(t * jnp.int32(16), 16)] = zeros16

            pltpu.sync_copy(o_v.at[pl.ds(0, SEQ_LEN)], out_hbm.at[r])

        @pl.when(d_scal == 0)
        def _():
            pltpu.sync_copy(seq_v, out_hbm.at[r])

    do_row(wid)
    do_row(wid + jnp.int32(32))


def kernel(input_ids):
    odd32 = input_ids[1::2].astype(jnp.int32)
    with jax.enable_x64(False):
        out_odd = _dropout_sc(odd32, jnp.asarray(_SIG1_NP), jnp.asarray(_SIG2_NP))
    return input_ids.at[1::2].set(out_odd.astype(input_ids.dtype))
